# baseline (speedup 1.0000x reference)
import functools

import jax
import jax.numpy as jnp
from jax import lax
from jax.experimental import pallas as pl
from jax.experimental.pallas import tpu as pltpu
from jax.experimental.pallas import tpu_sc as plsc

B = 4
S = 2048
D = 1024
NC = 2
NS = 16
NW = NC * NS
SPW = S // NW
CH = 16
NVEC = D // 16


def _sc_body(x_hbm, emb_hbm, out_hbm, ebuf, xbuf):
    wid = lax.axis_index("s") * NC + lax.axis_index("c")
    s0 = wid * SPW

    def chunk_body(c, carry):
        soff = s0 + c * CH
        pltpu.sync_copy(emb_hbm.at[pl.ds(soff, CH)], ebuf)
        for b in range(B):
            pltpu.sync_copy(x_hbm.at[b, pl.ds(soff, CH)], xbuf)

            def row_body(r, rc):
                for j in range(NVEC):
                    sl = pl.ds(j * 16, 16)
                    xbuf[r, sl] = xbuf[r, sl] + ebuf[r, sl]
                return rc

            lax.fori_loop(0, CH, row_body, 0)
            pltpu.sync_copy(xbuf, out_hbm.at[b, pl.ds(soff, CH)])
        return carry

    lax.fori_loop(0, SPW // CH, chunk_body, 0)


def kernel(x, emb_weight):
    mesh = plsc.VectorSubcoreMesh(core_axis_name="c", subcore_axis_name="s")
    f = pl.kernel(
        _sc_body,
        out_type=jax.ShapeDtypeStruct((B, S, D), jnp.float32),
        mesh=mesh,
        scratch_types=[
            pltpu.VMEM((CH, D), jnp.float32),
            pltpu.VMEM((CH, D), jnp.float32),
        ],
    )
    return f(x, emb_weight)


# speedup vs baseline: 1.3276x; 1.3276x over previous
import jax
import jax.numpy as jnp
from jax import lax
from jax.experimental import pallas as pl
from jax.experimental.pallas import tpu as pltpu
from jax.experimental.pallas import tpu_sc as plsc

B = 4
S = 2048
D = 1024
NC = 2
NS = 16
NW = NC * NS
SPW = S // NW
CH = 16
NCHUNK = SPW // CH
NSTAGE = NCHUNK * B
NVEC = D // 16


def _sc_body(x_hbm, emb_hbm, out_hbm,
             eb0, eb1, xb0, xb1,
             esem0, esem1, lsem0, lsem1, ssem0, ssem1):
    wid = lax.axis_index("s") * NC + lax.axis_index("c")
    s0 = wid * SPW
    ebufs, esems = (eb0, eb1), (esem0, esem1)
    xbufs, lsems, ssems = (xb0, xb1), (lsem0, lsem1), (ssem0, ssem1)

    def soff(c):
        return s0 + c * CH

    eload = {0: pltpu.async_copy(emb_hbm.at[pl.ds(soff(0), CH)], eb0, esem0)}
    xload = {0: pltpu.async_copy(x_hbm.at[0, pl.ds(soff(0), CH)], xb0, lsem0)}
    store = {}

    for t in range(NSTAGE):
        c, b = divmod(t, B)
        pb = t % 2
        if b == 0 and c + 1 < NCHUNK:
            ne = (c + 1) % 2
            eload[c + 1] = pltpu.async_copy(
                emb_hbm.at[pl.ds(soff(c + 1), CH)], ebufs[ne], esems[ne])
        if t + 1 < NSTAGE:
            if t - 1 in store:
                store.pop(t - 1).wait()
            nc_, nb_ = divmod(t + 1, B)
            np_ = (t + 1) % 2
            xload[t + 1] = pltpu.async_copy(
                x_hbm.at[nb_, pl.ds(soff(nc_), CH)], xbufs[np_], lsems[np_])
        xload.pop(t).wait()
        if b == 0:
            eload.pop(c).wait()

        xbuf, ebuf = xbufs[pb], ebufs[c % 2]

        def row_body(r, rc, xbuf=xbuf, ebuf=ebuf):
            for j in range(NVEC):
                sl = pl.ds(j * 16, 16)
                xbuf[r, sl] = xbuf[r, sl] + ebuf[r, sl]
            return rc

        lax.fori_loop(0, CH, row_body, 0)
        store[t] = pltpu.async_copy(
            xbuf, out_hbm.at[b, pl.ds(soff(c), CH)], ssems[pb])

    for h in store.values():
        h.wait()


def kernel(x, emb_weight):
    mesh = plsc.VectorSubcoreMesh(core_axis_name="c", subcore_axis_name="s")
    f = pl.kernel(
        _sc_body,
        out_type=jax.ShapeDtypeStruct((B, S, D), jnp.float32),
        mesh=mesh,
        scratch_types=[
            pltpu.VMEM((CH, D), jnp.float32),
            pltpu.VMEM((CH, D), jnp.float32),
            pltpu.VMEM((CH, D), jnp.float32),
            pltpu.VMEM((CH, D), jnp.float32),
            pltpu.SemaphoreType.DMA,
            pltpu.SemaphoreType.DMA,
            pltpu.SemaphoreType.DMA,
            pltpu.SemaphoreType.DMA,
            pltpu.SemaphoreType.DMA,
            pltpu.SemaphoreType.DMA,
        ],
    )
    return f(x, emb_weight)
